# baseline (speedup 1.0000x reference)
import functools

import jax
import jax.numpy as jnp
from jax import lax
from jax.experimental import pallas as pl
from jax.experimental.pallas import tpu as pltpu
from jax.experimental.pallas import tpu_sc as plsc

_B = 32
_H = 64
_W = 64
_N = _H * _W
_E = 2 * _N
_EREAL = _E - 2 * _W


def _edge_sort_body(x_ref, w_ref, eid_ref, key_ref, pay_ref):
    x = x_ref[...]
    colp = lax.broadcasted_iota(jnp.int32, (_B, _N), 1)
    inf = jnp.float32(jnp.inf)
    wh = jnp.where(colp % _W < _W - 1,
                   jnp.maximum(x, pltpu.roll(x, _N - 1, 1)), inf)
    wv = jnp.where(colp < _N - _W,
                   jnp.maximum(x, pltpu.roll(x, _N - _W, 1)), inf)
    key_ref[...] = jnp.concatenate([wh, wv], axis=1)
    iota = lax.broadcasted_iota(jnp.int32, (_B, _E), 1)
    pay_ref[...] = iota

    def stage(s, carry):
        k = jnp.int32(1) << s

        def cex(t, carry2):
            j = k >> (t + 1)
            key = key_ref[...]
            pay = pay_ref[...]
            low = (iota & j) == 0
            asc = (iota & k) == 0
            keep_small = jnp.logical_not(jnp.logical_xor(low, asc))
            sh_neg = jnp.int32(_E) - j
            pkey = jnp.where(low, pltpu.roll(key, sh_neg, 1),
                             pltpu.roll(key, j, 1))
            ppay = jnp.where(low, pltpu.roll(pay, sh_neg, 1),
                             pltpu.roll(pay, j, 1))
            swap = (keep_small & (key > pkey)) | (
                jnp.logical_not(keep_small) & (key < pkey))
            key_ref[...] = jnp.where(swap, pkey, key)
            pay_ref[...] = jnp.where(swap, ppay, pay)
            return carry2

        return lax.fori_loop(0, s, cex, carry)

    lax.fori_loop(1, 14, stage, jnp.int32(0))
    w_ref[...] = key_ref[...]
    eid_ref[...] = pay_ref[...]


def _edge_sort_tc(flat):
    return pl.pallas_call(
        _edge_sort_body,
        out_shape=(
            jax.ShapeDtypeStruct((_B, _E), jnp.float32),
            jax.ShapeDtypeStruct((_B, _E), jnp.int32),
        ),
        scratch_shapes=[
            pltpu.VMEM((_B, _E), jnp.float32),
            pltpu.VMEM((_B, _E), jnp.int32),
        ],
    )(flat)


def _uf_contrib(flat, w_sorted, eid_sorted):
    mesh = plsc.VectorSubcoreMesh(core_axis_name="c", subcore_axis_name="s")

    _NP = _N + 16
    _EP = _E + 16

    @functools.partial(
        pl.kernel,
        mesh=mesh,
        out_type=jax.ShapeDtypeStruct((_B, 16), jnp.float32),
        compiler_params=pltpu.CompilerParams(needs_layout_passes=False),
        scratch_types=[
            pltpu.VMEM((_NP,), jnp.float32),
            pltpu.VMEM((_EP,), jnp.float32),
            pltpu.VMEM((_EP,), jnp.int32),
            pltpu.VMEM((_NP,), jnp.int32),
            pltpu.VMEM((_NP,), jnp.float32),
            pltpu.VMEM((32,), jnp.int32),
            pltpu.VMEM((32,), jnp.int32),
            pltpu.VMEM((32,), jnp.float32),
            pltpu.VMEM((16,), jnp.float32),
        ],
    )
    def uf(vals_hbm, w_hbm, eid_hbm, out_hbm, vals_v, w_v, eid_v, parent_v,
           len_v, cru_v, crv_v, cw_v, row_v):
        b = lax.axis_index("s") * 2 + lax.axis_index("c")
        pltpu.sync_copy(vals_hbm.at[b], vals_v.at[pl.ds(0, _N)])
        pltpu.sync_copy(w_hbm.at[b], w_v.at[pl.ds(0, _E)])
        pltpu.sync_copy(eid_hbm.at[b], eid_v.at[pl.ds(0, _E)])

        lane = lax.iota(jnp.int32, 16)
        lane0 = lane == 0
        zeros16 = jnp.zeros((16,), jnp.float32)

        def sload(ref, i):
            return ref[pl.ds(i, 16)][0]

        def sstore(ref, i, v):
            plsc.store_scatter(ref, [jnp.full((16,), i, jnp.int32)],
                               jnp.full((16,), v), mask=lane0)

        def init_body(i, carry):
            parent_v[pl.ds(i * 16, 16)] = lane + i * 16
            len_v[pl.ds(i * 16, 16)] = zeros16
            return carry

        lax.fori_loop(0, _NP // 16, init_body, jnp.int32(0))

        def find(i):
            return lax.while_loop(lambda r: sload(parent_v, r) != r,
                                  lambda r: sload(parent_v, r), i)

        def vfind(r0):
            pr0 = plsc.load_gather(parent_v, [r0])

            def vcond(carry):
                r, pr = carry
                return plsc.all_reduce_population_count(pr != r)[0] > 0

            def vbody(carry):
                r, pr = carry
                r2 = jnp.where(pr != r, pr, r)
                return (r2, plsc.load_gather(parent_v, [r2]))

            r, _ = lax.while_loop(vcond, vbody, (r0, pr0))
            return r

        def chunk(i, cnt):
            wvec = w_v[pl.ds(i * 16, 16)]
            evec = eid_v[pl.ds(i * 16, 16)]
            uvec = evec & (_N - 1)
            vvec = uvec + jnp.where(evec >= _N, _W, 1)
            ruv = vfind(uvec)
            rvv = vfind(vvec)
            plsc.store_scatter(parent_v, [uvec], ruv)
            plsc.store_scatter(parent_v, [vvec], rvv)
            candm = ruv != rvv
            ncand = plsc.all_reduce_population_count(candm)[0]

            def have_cands(cnt):
                cru_v[pl.ds(0, 16)] = ruv
                crv_v[pl.ds(0, 16)] = rvv
                cw_v[pl.ds(0, 16)] = wvec

                def pcond(carry):
                    mask, cnt = carry
                    return plsc.all_reduce_population_count(mask)[0] > 0

                def pbody(carry):
                    mask, cnt = carry
                    d = plsc.all_reduce_ffs(mask)[0]
                    ru = find(sload(cru_v, d))
                    rv = find(sload(crv_v, d))

                    def merge_fn(cnt):
                        vru = sload(vals_v, ru)
                        vrv = sload(vals_v, rv)
                        ru_elder = (vru < vrv) | ((vru == vrv) & (ru < rv))
                        young = jnp.where(ru_elder, rv, ru)
                        elder = jnp.where(ru_elder, ru, rv)
                        sstore(parent_v, young, elder)
                        sstore(len_v, cnt,
                               sload(cw_v, d) - jnp.maximum(vru, vrv))
                        return cnt + 1

                    cnt = lax.cond(ru != rv, merge_fn, lambda c: c, cnt)
                    cs = lax.cumsum(mask.astype(jnp.int32))
                    return (mask & (cs != 1), cnt)

                _, cnt = lax.while_loop(pcond, pbody, (candm, cnt))
                return cnt

            return lax.cond(ncand > 0, have_cands, lambda c: c, cnt)

        lax.fori_loop(0, _EREAL // 16, chunk, jnp.int32(0))

        def topk_body(i, top):
            chunk16 = len_v[pl.ds(i * 16, 16)]
            cdesc = lax.rev(lax.sort(chunk16), (0,))
            return lax.sort(jnp.maximum(top, cdesc))

        top = lax.fori_loop(0, _NP // 16, topk_body,
                            jnp.zeros((16,), jnp.float32))

        sq = top * top
        contrib = jnp.where(lane == 15, 1.0 - sq,
                            jnp.where(lane >= 6, sq, zeros16))
        row_v[...] = contrib
        pltpu.sync_copy(row_v, out_hbm.at[b])

    return uf(flat, w_sorted, eid_sorted)


def kernel(data):
    assert data.shape == (_B, _H, _W), "check the shape!"
    flat = data.reshape(_B, _N)
    w_sorted, eid_sorted = _edge_sort_tc(flat)
    contrib = _uf_contrib(flat, w_sorted, eid_sorted)
    return jnp.sum(contrib)
